# Initial kernel scaffold; baseline (speedup 1.0000x reference)
import jax
import jax.numpy as jnp
from jax.experimental import pallas as pl


def kernel(boxes, scores):
    raise NotImplementedError("write your pallas kernel here")


# speedup vs baseline: 28.3933x; 28.3933x over previous
import jax
import jax.numpy as jnp
from jax.experimental import pallas as pl

_IOU_THRESHOLD = 0.5
_MAX_DET = 512
_LANES = 128
_NEG_INF = -1e30


def _nms_body(x1_ref, y1_ref, x2_ref, y2_ref, area_ref, sc_ref, out_ref):
    rows = sc_ref.shape[0]
    total = rows * _LANES
    idx2d = (jax.lax.broadcasted_iota(jnp.int32, (rows, _LANES), 0) * _LANES
             + jax.lax.broadcasted_iota(jnp.int32, (rows, _LANES), 1))
    lane = jax.lax.broadcasted_iota(jnp.int32, (1, _LANES), 1)

    x1 = x1_ref[...]
    y1 = y1_ref[...]
    x2 = x2_ref[...]
    y2 = y2_ref[...]
    area = area_ref[...]

    def body(i, ms):
        m = jnp.max(ms)
        valid = m > (_NEG_INF / 2)
        cand = jnp.where(ms == m, idx2d, jnp.int32(total))
        best = jnp.min(cand)
        r = best // _LANES
        c = best - r * _LANES
        onehot = lane == c

        def ext(plane_ref):
            row = plane_ref[pl.ds(r, 1), :]
            return jnp.sum(jnp.where(onehot, row, 0.0))

        bx1 = ext(x1_ref)
        by1 = ext(y1_ref)
        bx2 = ext(x2_ref)
        by2 = ext(y2_ref)

        ix1 = jnp.maximum(bx1, x1)
        iy1 = jnp.maximum(by1, y1)
        ix2 = jnp.minimum(bx2, x2)
        iy2 = jnp.minimum(by2, y2)
        inter = jnp.clip(ix2 - ix1, 0.0) * jnp.clip(iy2 - iy1, 0.0)
        area_a = (bx2 - bx1) * (by2 - by1)
        iou = inter / (area_a + area - inter + 1e-8)
        suppress = iou > _IOU_THRESHOLD
        ms_new = jnp.where(jnp.logical_and(valid, suppress),
                           jnp.float32(_NEG_INF), ms)

        valid_f = jnp.where(valid, jnp.float32(1.0), jnp.float32(0.0))
        out_row = (jnp.where(lane == 0, bx1, 0.0)
                   + jnp.where(lane == 1, by1, 0.0)
                   + jnp.where(lane == 2, bx2, 0.0)
                   + jnp.where(lane == 3, by2, 0.0)
                   + jnp.where(lane == 4, m, 0.0)) * valid_f
        out_ref[pl.ds(i, 1), :] = out_row[:, :5]
        return ms_new

    jax.lax.fori_loop(0, _MAX_DET, body, sc_ref[...])


def kernel(boxes, scores):
    n = boxes.shape[0]
    rows = (n + _LANES - 1) // _LANES
    rows = ((rows + 7) // 8) * 8
    padded = rows * _LANES
    pad = padded - n

    x1 = jnp.pad(boxes[:, 0], (0, pad)).reshape(rows, _LANES)
    y1 = jnp.pad(boxes[:, 1], (0, pad)).reshape(rows, _LANES)
    x2 = jnp.pad(boxes[:, 2], (0, pad)).reshape(rows, _LANES)
    y2 = jnp.pad(boxes[:, 3], (0, pad)).reshape(rows, _LANES)
    area = jnp.pad((boxes[:, 2] - boxes[:, 0]) * (boxes[:, 3] - boxes[:, 1]),
                   (0, pad)).reshape(rows, _LANES)
    sc = jnp.pad(scores, (0, pad), constant_values=_NEG_INF).reshape(rows, _LANES)

    return pl.pallas_call(
        _nms_body,
        out_shape=jax.ShapeDtypeStruct((_MAX_DET, 5), jnp.float32),
    )(x1, y1, x2, y2, area, sc)
